# Initial kernel scaffold; baseline (speedup 1.0000x reference)
import functools

import jax
import jax.numpy as jnp
from jax import lax
from jax.experimental import pallas as pl
from jax.experimental.pallas import tpu as pltpu
from jax.experimental.pallas import tpu_sc as plsc

N = 10000
E = 320000
F_IN = 128
F_HID = 256
F_OUT = 128
G = 128

NC = 2
NS = 16
EDGES_PER_W = E // (NC * NS)
BATCH_E = 80
NBATCH = EDGES_PER_W // BATCH_E
ROWS_PER_S = N // NS
ZCHUNK = 125
NZ = ROWS_PER_S // ZCHUNK

_mesh = plsc.VectorSubcoreMesh(core_axis_name="c", subcore_axis_name="s")


def _zero_vmem(ref, rows, width):
    zero = jnp.zeros((16,), jnp.float32)

    @pl.loop(0, rows)
    def _(i):
        @pl.loop(0, width, step=16)
        def _(j):
            ref[i, pl.ds(j, 16)] = zero


def _sc_degree(dst):
    @functools.partial(
        pl.kernel,
        mesh=_mesh,
        out_type=jax.ShapeDtypeStruct((NC * N, 16), jnp.float32),
        scratch_types=[
            pltpu.VMEM((BATCH_E,), jnp.int32),
            pltpu.VMEM((BATCH_E, 16), jnp.float32),
            pltpu.VMEM((ZCHUNK, 16), jnp.float32),
            pltpu.VMEM_SHARED((N, 16), jnp.float32),
        ],
    )
    def k(dst_hbm, out_hbm, idx_v, ones_v, z_v, acc):
        cid = lax.axis_index("c")
        sid = lax.axis_index("s")
        one = jnp.full((16,), 1.0, jnp.float32)

        @pl.loop(0, BATCH_E)
        def _(i):
            ones_v[i, pl.ds(0, 16)] = one

        _zero_vmem(z_v, ZCHUNK, 16)

        @pl.loop(0, NZ)
        def _(kk):
            pltpu.sync_copy(z_v, acc.at[pl.ds(sid * ROWS_PER_S + kk * ZCHUNK, ZCHUNK)])

        plsc.subcore_barrier()
        base = (cid * NS + sid) * EDGES_PER_W

        @pl.loop(0, NBATCH)
        def _(b):
            pltpu.sync_copy(dst_hbm.at[pl.ds(base + b * BATCH_E, BATCH_E)], idx_v)
            pltpu.sync_copy(ones_v, acc.at[idx_v], add=True)

        plsc.subcore_barrier()

        @pl.loop(0, NZ)
        def _(kk):
            r0 = sid * ROWS_PER_S + kk * ZCHUNK
            pltpu.sync_copy(acc.at[pl.ds(r0, ZCHUNK)], out_hbm.at[pl.ds(cid * N + r0, ZCHUNK)])

    return k(dst)


def _sc_propagate(table, src, dst):
    @functools.partial(
        pl.kernel,
        mesh=_mesh,
        out_type=jax.ShapeDtypeStruct((NC * N, F_OUT), jnp.float32),
        scratch_types=[
            pltpu.VMEM((BATCH_E,), jnp.int32),
            pltpu.VMEM((BATCH_E,), jnp.int32),
            pltpu.VMEM((BATCH_E, F_OUT), jnp.float32),
            pltpu.VMEM((ZCHUNK, F_OUT), jnp.float32),
            pltpu.VMEM_SHARED((N, F_OUT), jnp.float32),
            pltpu.SemaphoreType.DMA,
        ],
    )
    def k(tab_hbm, src_hbm, dst_hbm, out_hbm, si_v, di_v, rows_v, z_v, acc, sem):
        cid = lax.axis_index("c")
        sid = lax.axis_index("s")

        _zero_vmem(z_v, ZCHUNK, F_OUT)

        @pl.loop(0, NZ)
        def _(kk):
            pltpu.sync_copy(z_v, acc.at[pl.ds(sid * ROWS_PER_S + kk * ZCHUNK, ZCHUNK)])

        plsc.subcore_barrier()
        base = (cid * NS + sid) * EDGES_PER_W

        @pl.loop(0, NBATCH)
        def _(b):
            off = base + b * BATCH_E
            pltpu.sync_copy(src_hbm.at[pl.ds(off, BATCH_E)], si_v)
            pltpu.sync_copy(dst_hbm.at[pl.ds(off, BATCH_E)], di_v)
            pltpu.async_copy(tab_hbm.at[si_v], rows_v, sem).wait()
            pltpu.sync_copy(rows_v, acc.at[di_v], add=True)

        plsc.subcore_barrier()

        @pl.loop(0, NZ)
        def _(kk):
            r0 = sid * ROWS_PER_S + kk * ZCHUNK
            pltpu.sync_copy(acc.at[pl.ds(r0, ZCHUNK)], out_hbm.at[pl.ds(cid * N + r0, ZCHUNK)])

    return k(table, src, dst)


_HIGHEST = lax.Precision.HIGHEST


def _dot(a, b, dims):
    return lax.dot_general(a, b, (dims, ((), ())),
                           preferred_element_type=jnp.float32,
                           precision=_HIGHEST)


def _tc_stage_a(x, hist, root2d, W2b):
    def body(x_ref, h_ref, root_ref, w2b_ref, dinv_ref, xs_ref, rr_ref):
        indeg = h_ref[0:N, 0:1] + h_ref[N:2 * N, 0:1]
        dinv = lax.rsqrt(indeg + 1.0)
        dinv_ref[...] = dinv
        xs_ref[...] = x_ref[...] * dinv
        r = root_ref[...]
        col = lax.broadcasted_iota(jnp.int32, (G, N), 1)
        oh = (r == col).astype(jnp.float32)
        rootx = _dot(oh, x_ref[...], ((1,), (0,)))
        rr_ref[...] = _dot(jnp.maximum(rootx, 0.0), w2b_ref[...], ((1,), (0,)))

    return pl.pallas_call(
        body,
        out_shape=(
            jax.ShapeDtypeStruct((N, 1), jnp.float32),
            jax.ShapeDtypeStruct((N, F_IN), jnp.float32),
            jax.ShapeDtypeStruct((G, F_OUT), jnp.float32),
        ),
    )(x, hist, root2d, W2b)


RB = 1250
NRB = N // RB


def _tc_stage_b(s1, x, dinv, batch2d, W1, b1, W2a, rr):
    def body(s1a_ref, s1b_ref, x_ref, dinv_ref, bat_ref, w1_ref, b1_ref,
             w2a_ref, rr_ref, h1_ref, v_ref, vs_ref):
        dinv = dinv_ref[...]
        s1 = s1a_ref[...] + s1b_ref[...]
        xa = dinv * s1 + dinv * dinv * x_ref[...]
        h1 = _dot(xa, w1_ref[...], ((1,), (0,))) + b1_ref[...]
        u = jnp.maximum(h1, 0.0)
        col = lax.broadcasted_iota(jnp.int32, (RB, G), 1)
        oh = (bat_ref[...] == col).astype(jnp.float32)
        rrb = _dot(oh, rr_ref[...], ((1,), (0,)))
        v = _dot(u, w2a_ref[...], ((1,), (0,))) + rrb
        h1_ref[...] = h1
        v_ref[...] = v
        vs_ref[...] = dinv * v

    row = lambda i: (i, 0)
    rep = lambda i: (0, 0)
    return pl.pallas_call(
        body,
        grid=(NRB,),
        in_specs=[
            pl.BlockSpec((RB, F_OUT), row),
            pl.BlockSpec((RB, F_OUT), row),
            pl.BlockSpec((RB, F_IN), row),
            pl.BlockSpec((RB, 1), row),
            pl.BlockSpec((RB, 1), row),
            pl.BlockSpec((F_IN, F_HID), rep),
            pl.BlockSpec((1, F_HID), rep),
            pl.BlockSpec((F_HID, F_OUT), rep),
            pl.BlockSpec((G, F_OUT), rep),
        ],
        out_specs=(
            pl.BlockSpec((RB, F_HID), row),
            pl.BlockSpec((RB, F_OUT), row),
            pl.BlockSpec((RB, F_OUT), row),
        ),
        out_shape=(
            jax.ShapeDtypeStruct((N, F_HID), jnp.float32),
            jax.ShapeDtypeStruct((N, F_OUT), jnp.float32),
            jax.ShapeDtypeStruct((N, F_OUT), jnp.float32),
        ),
    )(s1[0:N], s1[N:2 * N], x, dinv, batch2d, W1, b1, W2a, rr)


def _tc_stage_c(s2, v, dinv, batch2d, h1, root2d, b2):
    def body(s2a_ref, s2b_ref, v_ref, dinv_ref, bat_ref, h1_ref, root_ref,
             b2_ref, out_ref, acc, cnt):
        i = pl.program_id(0)

        @pl.when(i == 0)
        def _():
            acc[...] = jnp.zeros_like(acc)
            cnt[...] = jnp.zeros_like(cnt)

        dinv = dinv_ref[...]
        h2 = dinv * (s2a_ref[...] + s2b_ref[...]) + dinv * dinv * v_ref[...]
        col = lax.broadcasted_iota(jnp.int32, (RB, G), 1)
        oh = (bat_ref[...] == col).astype(jnp.float32)
        acc[:, 0:F_OUT] += _dot(oh, h2, ((0,), (0,)))
        cnt[...] += jnp.sum(oh, axis=0)[:, None]
        rowid = lax.broadcasted_iota(jnp.int32, (G, RB), 1) + i * RB
        ohr = (root_ref[...] == rowid).astype(jnp.float32)
        acc[:, F_OUT:F_OUT + F_HID] += _dot(ohr, h1_ref[...], ((1,), (0,)))

        @pl.when(i == NRB - 1)
        def _():
            c = cnt[...]
            nonempty = c > 0.0
            pooled = acc[:, 0:F_OUT] / jnp.maximum(c, 1.0)
            pooled = pooled + jnp.where(nonempty, b2_ref[...], 0.0)
            out_ref[:, 0:F_OUT] = pooled
            out_ref[:, F_OUT:] = jnp.where(nonempty, acc[:, F_OUT:], 0.0)

    row = lambda i: (i, 0)
    rep = lambda i: (0, 0)
    return pl.pallas_call(
        body,
        grid=(NRB,),
        in_specs=[
            pl.BlockSpec((RB, F_OUT), row),
            pl.BlockSpec((RB, F_OUT), row),
            pl.BlockSpec((RB, F_OUT), row),
            pl.BlockSpec((RB, 1), row),
            pl.BlockSpec((RB, 1), row),
            pl.BlockSpec((RB, F_HID), row),
            pl.BlockSpec((G, 1), rep),
            pl.BlockSpec((1, F_OUT), rep),
        ],
        out_specs=pl.BlockSpec((G, F_OUT + F_HID), rep),
        out_shape=jax.ShapeDtypeStruct((G, F_OUT + F_HID), jnp.float32),
        scratch_shapes=[
            pltpu.VMEM((G, F_OUT + F_HID), jnp.float32),
            pltpu.VMEM((G, 1), jnp.float32),
        ],
    )(s2[0:N], s2[N:2 * N], v, dinv, batch2d, h1, root2d, b2)


def kernel(x, edge_index, batch, rootindex, W1, b1, W2, b2):
    x = x.astype(jnp.float32)
    src = edge_index[0]
    dst = edge_index[1]
    W2a = W2[0:F_HID]
    W2b = W2[F_HID:]
    root2d = rootindex.reshape(G, 1)
    batch2d = batch.reshape(N, 1)
    b1r = b1.reshape(1, F_HID)
    b2r = b2.reshape(1, F_OUT)

    hist = _sc_degree(dst)
    dinv, xs, rr = _tc_stage_a(x, hist, root2d, W2b)
    s1 = _sc_propagate(xs, src, dst)
    h1, v, vs = _tc_stage_b(s1, x, dinv, batch2d, W1, b1r, W2a, rr)
    s2 = _sc_propagate(vs, src, dst)
    return _tc_stage_c(s2, v, dinv, batch2d, h1, root2d, b2r)


# speedup vs baseline: 13.5113x; 13.5113x over previous
import functools

import jax
import jax.numpy as jnp
from jax import lax
from jax.experimental import pallas as pl
from jax.experimental.pallas import tpu as pltpu
from jax.experimental.pallas import tpu_sc as plsc

N = 10000
E = 320000
F_IN = 128
F_HID = 256
F_OUT = 128
G = 128

NC = 2
NS = 16
EDGES_PER_W = E // (NC * NS)
BATCH_E = 80
NBATCH = EDGES_PER_W // BATCH_E
NP = 10240
ROWS_PER_S = NP // NS
ZCHUNK = 128
NZ = ROWS_PER_S // ZCHUNK

@functools.cache
def _sc_mesh():
    return plsc.VectorSubcoreMesh(core_axis_name="c", subcore_axis_name="s")


def _zero_vmem(ref, rows, width):
    zero = jnp.zeros((16,), jnp.float32)

    @pl.loop(0, rows)
    def _(i):
        @pl.loop(0, width, step=16)
        def _(j):
            ref[i, pl.ds(j, 16)] = zero


def _sc_degree(dst):
    @functools.partial(
        pl.kernel,
        mesh=_sc_mesh(),
        out_type=jax.ShapeDtypeStruct((NC * NP, F_OUT), jnp.float32),
        scratch_types=[
            pltpu.VMEM((BATCH_E,), jnp.int32),
            pltpu.VMEM((BATCH_E, F_OUT), jnp.float32),
            pltpu.VMEM((ZCHUNK, F_OUT), jnp.float32),
            pltpu.VMEM_SHARED((NP, F_OUT), jnp.float32),
        ],
    )
    def k(dst_hbm, out_hbm, idx_v, ones_v, z_v, acc):
        cid = lax.axis_index("c")
        sid = lax.axis_index("s")
        one = jnp.full((16,), 1.0, jnp.float32)

        @pl.loop(0, BATCH_E)
        def _(i):
            @pl.loop(0, F_OUT, step=16)
            def _(j):
                ones_v[i, pl.ds(j, 16)] = one

        _zero_vmem(z_v, ZCHUNK, F_OUT)

        @pl.loop(0, NZ)
        def _(kk):
            pltpu.sync_copy(z_v, acc.at[pl.ds(sid * ROWS_PER_S + kk * ZCHUNK, ZCHUNK)])

        plsc.subcore_barrier()
        base = (cid * NS + sid) * EDGES_PER_W

        @pl.loop(0, NBATCH)
        def _(b):
            pltpu.sync_copy(dst_hbm.at[pl.ds(base + b * BATCH_E, BATCH_E)], idx_v)
            pltpu.sync_copy(ones_v, acc.at[idx_v], add=True)

        plsc.subcore_barrier()

        @pl.loop(0, NZ)
        def _(kk):
            r0 = sid * ROWS_PER_S + kk * ZCHUNK
            pltpu.sync_copy(acc.at[pl.ds(r0, ZCHUNK)], out_hbm.at[pl.ds(cid * NP + r0, ZCHUNK)])

    return k(dst)


def _sc_propagate(table, src, dst):
    @functools.partial(
        pl.kernel,
        mesh=_sc_mesh(),
        out_type=jax.ShapeDtypeStruct((NC * NP, F_OUT), jnp.float32),
        scratch_types=[
            pltpu.VMEM((BATCH_E,), jnp.int32),
            pltpu.VMEM((BATCH_E,), jnp.int32),
            pltpu.VMEM((BATCH_E, F_OUT), jnp.float32),
            pltpu.VMEM((ZCHUNK, F_OUT), jnp.float32),
            pltpu.VMEM_SHARED((NP, F_OUT), jnp.float32),
            pltpu.SemaphoreType.DMA,
        ],
    )
    def k(tab_hbm, src_hbm, dst_hbm, out_hbm, si_v, di_v, rows_v, z_v, acc, sem):
        cid = lax.axis_index("c")
        sid = lax.axis_index("s")

        _zero_vmem(z_v, ZCHUNK, F_OUT)

        @pl.loop(0, NZ)
        def _(kk):
            pltpu.sync_copy(z_v, acc.at[pl.ds(sid * ROWS_PER_S + kk * ZCHUNK, ZCHUNK)])

        plsc.subcore_barrier()
        base = (cid * NS + sid) * EDGES_PER_W

        @pl.loop(0, NBATCH)
        def _(b):
            off = base + b * BATCH_E
            pltpu.sync_copy(src_hbm.at[pl.ds(off, BATCH_E)], si_v)
            pltpu.sync_copy(dst_hbm.at[pl.ds(off, BATCH_E)], di_v)
            pltpu.async_copy(tab_hbm.at[si_v], rows_v, sem).wait()
            pltpu.sync_copy(rows_v, acc.at[di_v], add=True)

        plsc.subcore_barrier()

        @pl.loop(0, NZ)
        def _(kk):
            r0 = sid * ROWS_PER_S + kk * ZCHUNK
            pltpu.sync_copy(acc.at[pl.ds(r0, ZCHUNK)], out_hbm.at[pl.ds(cid * NP + r0, ZCHUNK)])

    return k(table, src, dst)


_HIGHEST = lax.Precision.HIGHEST


def _dot(a, b, dims):
    return lax.dot_general(a, b, (dims, ((), ())),
                           preferred_element_type=jnp.float32,
                           precision=_HIGHEST)


def _tc_stage_a(x, hist, root2d, W2b):
    def body(x_ref, h_ref, root_ref, w2b_ref, dinv_ref, xs_ref, rr_ref):
        indeg = h_ref[0:N, 0:1] + h_ref[NP:NP + N, 0:1]
        dinv = lax.rsqrt(indeg + 1.0)
        dinv_ref[...] = dinv
        xs_ref[...] = x_ref[...] * dinv
        r = root_ref[...]
        col = lax.broadcasted_iota(jnp.int32, (G, N), 1)
        oh = (r == col).astype(jnp.float32)
        rootx = _dot(oh, x_ref[...], ((1,), (0,)))
        rr_ref[...] = _dot(jnp.maximum(rootx, 0.0), w2b_ref[...], ((1,), (0,)))

    return pl.pallas_call(
        body,
        out_shape=(
            jax.ShapeDtypeStruct((N, 1), jnp.float32),
            jax.ShapeDtypeStruct((N, F_IN), jnp.float32),
            jax.ShapeDtypeStruct((G, F_OUT), jnp.float32),
        ),
    )(x, hist, root2d, W2b)


RB = 2000
NRB = N // RB


def _tc_stage_b(s1, x, dinv, batch2d, W1, b1, W2a, rr):
    def body(s1a_ref, s1b_ref, x_ref, dinv_ref, bat_ref, w1_ref, b1_ref,
             w2a_ref, rr_ref, h1_ref, v_ref, vs_ref):
        dinv = dinv_ref[...]
        s1 = s1a_ref[...] + s1b_ref[...]
        xa = dinv * s1 + dinv * dinv * x_ref[...]
        h1 = _dot(xa, w1_ref[...], ((1,), (0,))) + b1_ref[...]
        u = jnp.maximum(h1, 0.0)
        col = lax.broadcasted_iota(jnp.int32, (RB, G), 1)
        oh = (bat_ref[...] == col).astype(jnp.float32)
        rrb = _dot(oh, rr_ref[...], ((1,), (0,)))
        v = _dot(u, w2a_ref[...], ((1,), (0,))) + rrb
        h1_ref[...] = h1
        v_ref[...] = v
        vs_ref[...] = dinv * v

    row = lambda i: (i, 0)
    rep = lambda i: (0, 0)
    return pl.pallas_call(
        body,
        grid=(NRB,),
        in_specs=[
            pl.BlockSpec((RB, F_OUT), row),
            pl.BlockSpec((RB, F_OUT), row),
            pl.BlockSpec((RB, F_IN), row),
            pl.BlockSpec((RB, 1), row),
            pl.BlockSpec((RB, 1), row),
            pl.BlockSpec((F_IN, F_HID), rep),
            pl.BlockSpec((1, F_HID), rep),
            pl.BlockSpec((F_HID, F_OUT), rep),
            pl.BlockSpec((G, F_OUT), rep),
        ],
        out_specs=(
            pl.BlockSpec((RB, F_HID), row),
            pl.BlockSpec((RB, F_OUT), row),
            pl.BlockSpec((RB, F_OUT), row),
        ),
        out_shape=(
            jax.ShapeDtypeStruct((N, F_HID), jnp.float32),
            jax.ShapeDtypeStruct((N, F_OUT), jnp.float32),
            jax.ShapeDtypeStruct((N, F_OUT), jnp.float32),
        ),
    )(s1[0:N], s1[NP:NP + N], x, dinv, batch2d, W1, b1, W2a, rr)


def _tc_stage_c(s2, v, dinv, batch2d, h1, root2d, b2):
    def body(s2a_ref, s2b_ref, v_ref, dinv_ref, bat_ref, h1_ref, root_ref,
             b2_ref, out_ref, acc, cnt):
        i = pl.program_id(0)

        @pl.when(i == 0)
        def _():
            acc[...] = jnp.zeros_like(acc)
            cnt[...] = jnp.zeros_like(cnt)

        dinv = dinv_ref[...]
        h2 = dinv * (s2a_ref[...] + s2b_ref[...]) + dinv * dinv * v_ref[...]
        col = lax.broadcasted_iota(jnp.int32, (RB, G), 1)
        oh = (bat_ref[...] == col).astype(jnp.float32)
        acc[:, 0:F_OUT] += _dot(oh, h2, ((0,), (0,)))
        cnt[...] += jnp.sum(oh, axis=0)[:, None]
        rowid = lax.broadcasted_iota(jnp.int32, (G, RB), 1) + i * RB
        ohr = (root_ref[...] == rowid).astype(jnp.float32)
        acc[:, F_OUT:F_OUT + F_HID] += _dot(ohr, h1_ref[...], ((1,), (0,)))

        @pl.when(i == NRB - 1)
        def _():
            c = cnt[...]
            nonempty = c > 0.0
            pooled = acc[:, 0:F_OUT] / jnp.maximum(c, 1.0)
            pooled = pooled + jnp.where(nonempty, b2_ref[...], 0.0)
            out_ref[:, 0:F_OUT] = pooled
            out_ref[:, F_OUT:] = jnp.where(nonempty, acc[:, F_OUT:], 0.0)

    row = lambda i: (i, 0)
    rep = lambda i: (0, 0)
    return pl.pallas_call(
        body,
        grid=(NRB,),
        in_specs=[
            pl.BlockSpec((RB, F_OUT), row),
            pl.BlockSpec((RB, F_OUT), row),
            pl.BlockSpec((RB, F_OUT), row),
            pl.BlockSpec((RB, 1), row),
            pl.BlockSpec((RB, 1), row),
            pl.BlockSpec((RB, F_HID), row),
            pl.BlockSpec((G, 1), rep),
            pl.BlockSpec((1, F_OUT), rep),
        ],
        out_specs=pl.BlockSpec((G, F_OUT + F_HID), rep),
        out_shape=jax.ShapeDtypeStruct((G, F_OUT + F_HID), jnp.float32),
        scratch_shapes=[
            pltpu.VMEM((G, F_OUT + F_HID), jnp.float32),
            pltpu.VMEM((G, 1), jnp.float32),
        ],
    )(s2[0:N], s2[NP:NP + N], v, dinv, batch2d, h1, root2d, b2)


def kernel(x, edge_index, batch, rootindex, W1, b1, W2, b2):
    x = x.astype(jnp.float32)
    src = edge_index[0]
    dst = edge_index[1]
    W2a = W2[0:F_HID]
    W2b = W2[F_HID:]
    root2d = rootindex.reshape(G, 1)
    batch2d = batch.reshape(N, 1)
    b1r = b1.reshape(1, F_HID)
    b2r = b2.reshape(1, F_OUT)

    hist = _sc_degree(dst)
    dinv, xs, rr = _tc_stage_a(x, hist, root2d, W2b)
    s1 = _sc_propagate(xs, src, dst)
    h1, v, vs = _tc_stage_b(s1, x, dinv, batch2d, W1, b1r, W2a, rr)
    s2 = _sc_propagate(vs, src, dst)
    return _tc_stage_c(s2, v, dinv, batch2d, h1, root2d, b2r)


# speedup vs baseline: 22.6303x; 1.6749x over previous
import functools

import jax
import jax.numpy as jnp
from jax import lax
from jax.experimental import pallas as pl
from jax.experimental.pallas import tpu as pltpu
from jax.experimental.pallas import tpu_sc as plsc

N = 10000
E = 320000
F_IN = 128
F_HID = 256
F_OUT = 128
G = 128

NC = 2
NS = 16
EDGES_PER_W = E // (NC * NS)
BATCH_E = 80
NBATCH = EDGES_PER_W // BATCH_E
NP = 10240
ROWS_PER_S = NP // NS
ZCHUNK = 128
NZ = ROWS_PER_S // ZCHUNK

@functools.cache
def _sc_mesh():
    return plsc.VectorSubcoreMesh(core_axis_name="c", subcore_axis_name="s")


def _zero_vmem(ref, rows, width):
    zero = jnp.zeros((16,), jnp.float32)

    @pl.loop(0, rows)
    def _(i):
        @pl.loop(0, width, step=16)
        def _(j):
            ref[i, pl.ds(j, 16)] = zero


def _sc_degree(dst):
    @functools.partial(
        pl.kernel,
        mesh=_sc_mesh(),
        out_type=jax.ShapeDtypeStruct((NC * NP, F_OUT), jnp.float32),
        scratch_types=[
            pltpu.VMEM((NBATCH, BATCH_E), jnp.int32),
            pltpu.VMEM((BATCH_E, F_OUT), jnp.float32),
            pltpu.VMEM((ZCHUNK, F_OUT), jnp.float32),
            pltpu.VMEM_SHARED((NP, F_OUT), jnp.float32),
        ],
    )
    def k(dst_hbm, out_hbm, di_all, ones_v, z_v, acc):
        cid = lax.axis_index("c")
        sid = lax.axis_index("s")
        wid = cid * NS + sid
        one = jnp.full((16,), 1.0, jnp.float32)

        pltpu.sync_copy(dst_hbm.at[wid], di_all)

        @pl.loop(0, BATCH_E)
        def _(i):
            @pl.loop(0, F_OUT, step=16)
            def _(j):
                ones_v[i, pl.ds(j, 16)] = one

        _zero_vmem(z_v, ZCHUNK, F_OUT)

        @pl.loop(0, NZ)
        def _(kk):
            pltpu.sync_copy(z_v, acc.at[pl.ds(sid * ROWS_PER_S + kk * ZCHUNK, ZCHUNK)])

        plsc.subcore_barrier()

        @pl.loop(0, NBATCH)
        def _(b):
            pltpu.sync_copy(ones_v, acc.at[di_all.at[b]], add=True)

        plsc.subcore_barrier()

        @pl.loop(0, NZ)
        def _(kk):
            r0 = sid * ROWS_PER_S + kk * ZCHUNK
            pltpu.sync_copy(acc.at[pl.ds(r0, ZCHUNK)], out_hbm.at[pl.ds(cid * NP + r0, ZCHUNK)])

    return k(dst)


NBUF = 4
NB_MAIN = (NBATCH // NBUF) * NBUF


def _sc_propagate(table, src, dst):
    @functools.partial(
        pl.kernel,
        mesh=_sc_mesh(),
        out_type=jax.ShapeDtypeStruct((NC * NP, F_OUT), jnp.float32),
        scratch_types=[
            pltpu.VMEM((2 * NBUF, BATCH_E), jnp.int32),
        ] + [pltpu.VMEM((BATCH_E, F_OUT), jnp.float32)] * NBUF + [
            pltpu.VMEM_SHARED((NP, F_OUT), jnp.float32),
        ] + [pltpu.SemaphoreType.DMA] * (NBUF + 1),
    )
    def k(tab_hbm, src_hbm, dst_hbm, out_hbm, idxb, *rest):
        bufs = rest[:NBUF]
        acc = rest[NBUF]
        isem = rest[NBUF + 1]
        gsems = rest[NBUF + 2:]
        cid = lax.axis_index("c")
        sid = lax.axis_index("s")
        base = (cid * NS + sid) * EDGES_PER_W

        _zero_vmem(bufs[0], BATCH_E, F_OUT)

        @pl.loop(0, ROWS_PER_S // BATCH_E)
        def _(kk):
            pltpu.sync_copy(bufs[0], acc.at[pl.ds(sid * ROWS_PER_S + kk * BATCH_E, BATCH_E)])

        plsc.subcore_barrier()

        def do_group(b, width):
            hidx = []
            for j in range(width):
                off = base + (b + j) * BATCH_E
                hidx.append(pltpu.async_copy(
                    src_hbm.at[pl.ds(off, BATCH_E)], idxb.at[j], isem))
                hidx.append(pltpu.async_copy(
                    dst_hbm.at[pl.ds(off, BATCH_E)], idxb.at[NBUF + j], isem))
            for h in hidx:
                h.wait()
            hg = [pltpu.async_copy(tab_hbm.at[idxb.at[j]], bufs[j], gsems[j])
                  for j in range(width)]
            for j in range(width):
                hg[j].wait()
                pltpu.sync_copy(bufs[j], acc.at[idxb.at[NBUF + j]], add=True)

        @pl.loop(0, NB_MAIN, step=NBUF)
        def _(b):
            do_group(b, NBUF)

        for b in range(NB_MAIN, NBATCH):
            do_group(b, 1)

        plsc.subcore_barrier()
        r0 = sid * ROWS_PER_S
        pltpu.sync_copy(acc.at[pl.ds(r0, ROWS_PER_S)],
                        out_hbm.at[pl.ds(cid * NP + r0, ROWS_PER_S)])

    return k(table, src, dst)


_HIGHEST = lax.Precision.HIGHEST


def _dot(a, b, dims):
    return lax.dot_general(a, b, (dims, ((), ())),
                           preferred_element_type=jnp.float32,
                           precision=_HIGHEST)


def _tc_stage_a(x, hist, root2d, W2b):
    def body(x_ref, h_ref, root_ref, w2b_ref, dinv_ref, xs_ref, rr_ref):
        indeg = h_ref[0:N, 0:1] + h_ref[NP:NP + N, 0:1]
        dinv = lax.rsqrt(indeg + 1.0)
        dinv_ref[...] = dinv
        xs_ref[...] = x_ref[...] * dinv
        r = root_ref[...]
        col = lax.broadcasted_iota(jnp.int32, (G, N), 1)
        oh = (r == col).astype(jnp.float32)
        rootx = _dot(oh, x_ref[...], ((1,), (0,)))
        rr_ref[...] = _dot(jnp.maximum(rootx, 0.0), w2b_ref[...], ((1,), (0,)))

    return pl.pallas_call(
        body,
        out_shape=(
            jax.ShapeDtypeStruct((N, 1), jnp.float32),
            jax.ShapeDtypeStruct((N, F_IN), jnp.float32),
            jax.ShapeDtypeStruct((G, F_OUT), jnp.float32),
        ),
    )(x, hist, root2d, W2b)


RB = 2000
NRB = N // RB


def _tc_stage_b(s1, x, dinv, batch2d, W1, b1, W2a, rr):
    def body(s1a_ref, s1b_ref, x_ref, dinv_ref, bat_ref, w1_ref, b1_ref,
             w2a_ref, rr_ref, h1_ref, v_ref, vs_ref):
        dinv = dinv_ref[...]
        s1 = s1a_ref[...] + s1b_ref[...]
        xa = dinv * s1 + dinv * dinv * x_ref[...]
        h1 = _dot(xa, w1_ref[...], ((1,), (0,))) + b1_ref[...]
        u = jnp.maximum(h1, 0.0)
        col = lax.broadcasted_iota(jnp.int32, (RB, G), 1)
        oh = (bat_ref[...] == col).astype(jnp.float32)
        rrb = _dot(oh, rr_ref[...], ((1,), (0,)))
        v = _dot(u, w2a_ref[...], ((1,), (0,))) + rrb
        h1_ref[...] = h1
        v_ref[...] = v
        vs_ref[...] = dinv * v

    row = lambda i: (i, 0)
    rep = lambda i: (0, 0)
    return pl.pallas_call(
        body,
        grid=(NRB,),
        in_specs=[
            pl.BlockSpec((RB, F_OUT), row),
            pl.BlockSpec((RB, F_OUT), row),
            pl.BlockSpec((RB, F_IN), row),
            pl.BlockSpec((RB, 1), row),
            pl.BlockSpec((RB, 1), row),
            pl.BlockSpec((F_IN, F_HID), rep),
            pl.BlockSpec((1, F_HID), rep),
            pl.BlockSpec((F_HID, F_OUT), rep),
            pl.BlockSpec((G, F_OUT), rep),
        ],
        out_specs=(
            pl.BlockSpec((RB, F_HID), row),
            pl.BlockSpec((RB, F_OUT), row),
            pl.BlockSpec((RB, F_OUT), row),
        ),
        out_shape=(
            jax.ShapeDtypeStruct((N, F_HID), jnp.float32),
            jax.ShapeDtypeStruct((N, F_OUT), jnp.float32),
            jax.ShapeDtypeStruct((N, F_OUT), jnp.float32),
        ),
    )(s1[0:N], s1[NP:NP + N], x, dinv, batch2d, W1, b1, W2a, rr)


def _tc_stage_c(s2, v, dinv, batch2d, h1, root2d, b2):
    def body(s2a_ref, s2b_ref, v_ref, dinv_ref, bat_ref, h1_ref, root_ref,
             b2_ref, out_ref, acc, cnt):
        i = pl.program_id(0)

        @pl.when(i == 0)
        def _():
            acc[...] = jnp.zeros_like(acc)
            cnt[...] = jnp.zeros_like(cnt)

        dinv = dinv_ref[...]
        h2 = dinv * (s2a_ref[...] + s2b_ref[...]) + dinv * dinv * v_ref[...]
        col = lax.broadcasted_iota(jnp.int32, (RB, G), 1)
        oh = (bat_ref[...] == col).astype(jnp.float32)
        acc[:, 0:F_OUT] += _dot(oh, h2, ((0,), (0,)))
        cnt[...] += jnp.sum(oh, axis=0)[:, None]
        rowid = lax.broadcasted_iota(jnp.int32, (G, RB), 1) + i * RB
        ohr = (root_ref[...] == rowid).astype(jnp.float32)
        acc[:, F_OUT:F_OUT + F_HID] += _dot(ohr, h1_ref[...], ((1,), (0,)))

        @pl.when(i == NRB - 1)
        def _():
            c = cnt[...]
            nonempty = c > 0.0
            pooled = acc[:, 0:F_OUT] / jnp.maximum(c, 1.0)
            pooled = pooled + jnp.where(nonempty, b2_ref[...], 0.0)
            out_ref[:, 0:F_OUT] = pooled
            out_ref[:, F_OUT:] = jnp.where(nonempty, acc[:, F_OUT:], 0.0)

    row = lambda i: (i, 0)
    rep = lambda i: (0, 0)
    return pl.pallas_call(
        body,
        grid=(NRB,),
        in_specs=[
            pl.BlockSpec((RB, F_OUT), row),
            pl.BlockSpec((RB, F_OUT), row),
            pl.BlockSpec((RB, F_OUT), row),
            pl.BlockSpec((RB, 1), row),
            pl.BlockSpec((RB, 1), row),
            pl.BlockSpec((RB, F_HID), row),
            pl.BlockSpec((G, 1), rep),
            pl.BlockSpec((1, F_OUT), rep),
        ],
        out_specs=pl.BlockSpec((G, F_OUT + F_HID), rep),
        out_shape=jax.ShapeDtypeStruct((G, F_OUT + F_HID), jnp.float32),
        scratch_shapes=[
            pltpu.VMEM((G, F_OUT + F_HID), jnp.float32),
            pltpu.VMEM((G, 1), jnp.float32),
        ],
    )(s2[0:N], s2[NP:NP + N], v, dinv, batch2d, h1, root2d, b2)


def kernel(x, edge_index, batch, rootindex, W1, b1, W2, b2):
    x = x.astype(jnp.float32)
    src = edge_index[0]
    dst = edge_index[1]
    W2a = W2[0:F_HID]
    W2b = W2[F_HID:]
    root2d = rootindex.reshape(G, 1)
    batch2d = batch.reshape(N, 1)
    b1r = b1.reshape(1, F_HID)
    b2r = b2.reshape(1, F_OUT)

    dst3 = dst.reshape(NC * NS, NBATCH, BATCH_E)
    hist = _sc_degree(dst3)
    dinv, xs, rr = _tc_stage_a(x, hist, root2d, W2b)
    s1 = _sc_propagate(xs, src, dst)
    h1, v, vs = _tc_stage_b(s1, x, dinv, batch2d, W1, b1r, W2a, rr)
    s2 = _sc_propagate(vs, src, dst)
    return _tc_stage_c(s2, v, dinv, batch2d, h1, root2d, b2r)
